# baseline (speedup 1.0000x reference)
import math

import jax
import jax.numpy as jnp
from jax import lax
from jax.experimental import pallas as pl
from jax.experimental.pallas import tpu as pltpu
from jax.experimental.pallas import tpu_sc as plsc

NC, NS, L = 2, 16, 16
NW = NC * NS
B, S, D = 4096, 200, 64
RPW = B * S // NW
C = 128
NCH = RPW // C
G = D // L
NBUF = 5


def _pos_encoding():
    position = jnp.arange(0, S, dtype=jnp.float32)[:, None]
    div_term = jnp.exp(
        jnp.arange(0, D, 2, dtype=jnp.float32) * -(math.log(10000.0) / D))
    ang = position * div_term
    pe = jnp.zeros((S, D), dtype=jnp.float32)
    pe = pe.at[:, 0::2].set(jnp.sin(ang))
    pe = pe.at[:, 1::2].set(jnp.cos(ang))
    return pe


def _embed_body(ids_hbm, table_hbm, pos_hbm, out_hbm, idx_v, pos_v,
                in_bufs, out_bufs, in_sems, out_sems):
    wid = lax.axis_index("s") * NC + lax.axis_index("c")
    pltpu.sync_copy(ids_hbm.at[wid], idx_v)
    pltpu.sync_copy(pos_hbm, pos_v)

    def gather(c, b):
        pltpu.async_copy(table_hbm.at[idx_v.at[c]], in_bufs[b], in_sems[b])

    def write_out(c, b):
        pltpu.async_copy(out_bufs[b], out_hbm.at[wid, c], out_sems[b])

    for b in range(NBUF):
        gather(b, b)

    @pl.loop(0, NCH // NBUF)
    def _lap(lap):
        for b in range(NBUF):
            c = lap * NBUF + b
            pltpu.make_async_copy(
                table_hbm.at[idx_v.at[c]], in_bufs[b], in_sems[b]).wait()

            @pl.when(c >= NBUF)
            def _():
                pltpu.make_async_copy(
                    out_bufs[b], out_hbm.at[wid, c], out_sems[b]).wait()

            base = (wid * RPW + c * C) % S

            @pl.loop(0, C, unroll=4)
            def _row(r):
                p = lax.rem(base + r, S)
                for g in range(G):
                    out_bufs[b][r, pl.ds(g * L, L)] = (
                        in_bufs[b][r, pl.ds(g * L, L)]
                        + pos_v[p, pl.ds(g * L, L)])

            @pl.when(c + NBUF < NCH)
            def _():
                gather(c + NBUF, b)

            write_out(c, b)

    for b in range(NBUF):
        pltpu.make_async_copy(
            out_bufs[b], out_hbm.at[wid, 0], out_sems[b]).wait()


def kernel(input_ids, token_embedding_weight):
    ids_r = input_ids.reshape(NW, NCH, C)
    pos = _pos_encoding()
    mesh = plsc.VectorSubcoreMesh(
        core_axis_name="c", subcore_axis_name="s",
        num_cores=NC, num_subcores=NS)
    f = pl.kernel(
        _embed_body,
        out_type=jax.ShapeDtypeStruct((NW, NCH, C, D), jnp.float32),
        mesh=mesh,
        scratch_types=[
            pltpu.VMEM((NCH, C), jnp.int32),
            pltpu.VMEM((S, D), jnp.float32),
            [pltpu.VMEM((C, D), jnp.float32) for _ in range(NBUF)],
            [pltpu.VMEM((C, D), jnp.float32) for _ in range(NBUF)],
            [pltpu.SemaphoreType.DMA for _ in range(NBUF)],
            [pltpu.SemaphoreType.DMA for _ in range(NBUF)],
        ],
        compiler_params=pltpu.CompilerParams(use_tc_tiling_on_sc=False),
    )
    out = f(ids_r, token_embedding_weight, pos)
    return out.reshape(B, S, D)


# speedup vs baseline: 1.2280x; 1.2280x over previous
import math

import jax
import jax.numpy as jnp
from jax import lax
from jax.experimental import pallas as pl
from jax.experimental.pallas import tpu as pltpu
from jax.experimental.pallas import tpu_sc as plsc

NC, NS, L = 2, 16, 16
NW = NC * NS
B, S, D = 4096, 200, 64
SEQ_PER_W = B // NW
G = D // L


def _pos_encoding():
    position = jnp.arange(0, S, dtype=jnp.float32)[:, None]
    div_term = jnp.exp(
        jnp.arange(0, D, 2, dtype=jnp.float32) * -(math.log(10000.0) / D))
    ang = position * div_term
    pe = jnp.zeros((S, D), dtype=jnp.float32)
    pe = pe.at[:, 0::2].set(jnp.sin(ang))
    pe = pe.at[:, 1::2].set(jnp.cos(ang))
    return pe


def _embed_body(ids_hbm, table_hbm, pos_hbm, out_hbm, idx_v, pos_v,
                buf0, buf1, sem):
    wid = lax.axis_index("s") * NC + lax.axis_index("c")
    base_seq = wid * SEQ_PER_W
    pltpu.sync_copy(ids_hbm.at[wid], idx_v)
    pltpu.sync_copy(pos_hbm, pos_v)

    bufs = [buf0, buf1]

    def gather(p, b):
        pltpu.async_copy(table_hbm.at[idx_v.at[p]], bufs[b], sem)

    gather(0, 0)

    @pl.loop(0, S // 2)
    def _lap(lap):
        for h in range(2):
            p = lap * 2 + h
            pltpu.make_async_copy(
                table_hbm.at[idx_v.at[p]], bufs[h], sem).wait()

            @pl.when(p + 1 < S)
            def _():
                gather(p + 1, 1 - h)

            pvals = [pos_v[p, pl.ds(g * L, L)] for g in range(G)]

            @pl.loop(0, SEQ_PER_W, unroll=8)
            def _row(r):
                for g in range(G):
                    bufs[h][r, pl.ds(g * L, L)] = (
                        bufs[h][r, pl.ds(g * L, L)] + pvals[g])

            pltpu.sync_copy(
                bufs[h], out_hbm.at[pl.ds(base_seq, SEQ_PER_W), p])


def kernel(input_ids, token_embedding_weight):
    ids_t = input_ids.reshape(NW, SEQ_PER_W, S).transpose(0, 2, 1)
    pos = _pos_encoding()
    mesh = plsc.VectorSubcoreMesh(
        core_axis_name="c", subcore_axis_name="s",
        num_cores=NC, num_subcores=NS)
    f = pl.kernel(
        _embed_body,
        out_type=jax.ShapeDtypeStruct((B, S, D), jnp.float32),
        mesh=mesh,
        scratch_types=[
            pltpu.VMEM((S, SEQ_PER_W), jnp.int32),
            pltpu.VMEM((S, D), jnp.float32),
            pltpu.VMEM((SEQ_PER_W, D), jnp.float32),
            pltpu.VMEM((SEQ_PER_W, D), jnp.float32),
            pltpu.SemaphoreType.DMA,
        ],
        compiler_params=pltpu.CompilerParams(use_tc_tiling_on_sc=False),
    )
    return f(ids_t, token_embedding_weight, pos)
